# Initial kernel scaffold; baseline (speedup 1.0000x reference)
import jax
import jax.numpy as jnp
from jax.experimental import pallas as pl


def kernel(pos_seq, pos_emb):
    raise NotImplementedError("write your pallas kernel here")


# speedup vs baseline: 2.4439x; 2.4439x over previous
import functools

import jax
import jax.numpy as jnp
from jax import lax
from jax.experimental import pallas as pl
from jax.experimental.pallas import tpu as pltpu
from jax.experimental.pallas import tpu_sc as plsc

_DEMB = 768
_NC = 2
_NS = 16
_NW = _NC * _NS
_B = 32768
_BPW = _B // _NW
_C = 64
_NCHUNK = _BPW // _C

_mesh = plsc.VectorSubcoreMesh(core_axis_name="c", subcore_axis_name="s")


@functools.partial(
    pl.kernel,
    out_type=jax.ShapeDtypeStruct((_B, _DEMB), jnp.float32),
    mesh=_mesh,
    scratch_types=[
        pltpu.VMEM((_BPW,), jnp.int32),
        pltpu.VMEM((2, _C, _DEMB), jnp.float32),
        pltpu.SemaphoreType.DMA,
        pltpu.SemaphoreType.DMA,
    ],
)
def _emb_gather(idx_hbm, table_hbm, out_hbm, idx_v, rows_v, gsem, ssem):
    wid = lax.axis_index("s") * _NC + lax.axis_index("c")
    base = wid * _BPW
    pltpu.sync_copy(idx_hbm.at[pl.ds(base, _BPW)], idx_v)

    pltpu.async_copy(table_hbm.at[idx_v.at[pl.ds(0, _C)]], rows_v.at[0], gsem)
    for c in range(_NCHUNK):
        cur = c % 2
        pltpu.make_async_copy(
            table_hbm.at[idx_v.at[pl.ds(c * _C, _C)]], rows_v.at[cur], gsem
        ).wait()
        if c + 1 < _NCHUNK:
            if c >= 1:
                pltpu.make_async_copy(
                    rows_v.at[1 - cur],
                    out_hbm.at[pl.ds(base + (c - 1) * _C, _C)],
                    ssem,
                ).wait()
            pltpu.async_copy(
                table_hbm.at[idx_v.at[pl.ds((c + 1) * _C, _C)]],
                rows_v.at[1 - cur],
                gsem,
            )
        pltpu.async_copy(
            rows_v.at[cur], out_hbm.at[pl.ds(base + c * _C, _C)], ssem
        )
    for c in (_NCHUNK - 2, _NCHUNK - 1):
        pltpu.make_async_copy(
            rows_v.at[c % 2], out_hbm.at[pl.ds(base + c * _C, _C)], ssem
        ).wait()


def kernel(pos_seq, pos_emb):
    idx = pos_seq.reshape(-1).astype(jnp.int32)
    out = _emb_gather(idx, pos_emb)
    return out.reshape(pos_seq.shape + (pos_emb.shape[-1],))


# speedup vs baseline: 2.4762x; 1.0132x over previous
import functools

import jax
import jax.numpy as jnp
from jax import lax
from jax.experimental import pallas as pl
from jax.experimental.pallas import tpu as pltpu
from jax.experimental.pallas import tpu_sc as plsc

_DEMB = 768
_NC = 2
_NS = 16
_NW = _NC * _NS
_B = 32768
_BPW = _B // _NW
_C = 32
_NBUF = 4
_NCHUNK = _BPW // _C

_mesh = plsc.VectorSubcoreMesh(core_axis_name="c", subcore_axis_name="s")


@functools.partial(
    pl.kernel,
    out_type=jax.ShapeDtypeStruct((_B, _DEMB), jnp.float32),
    mesh=_mesh,
    scratch_types=[
        pltpu.VMEM((_BPW,), jnp.int32),
        pltpu.VMEM((_NBUF, _C, _DEMB), jnp.float32),
        pltpu.SemaphoreType.DMA((_NBUF,)),
        pltpu.SemaphoreType.DMA((_NBUF,)),
    ],
)
def _emb_gather(idx_hbm, table_hbm, out_hbm, idx_v, rows_v, gsem, ssem):
    wid = lax.axis_index("s") * _NC + lax.axis_index("c")
    base = wid * _BPW
    pltpu.sync_copy(idx_hbm.at[pl.ds(base, _BPW)], idx_v)

    def gather(c):
        b = c % _NBUF
        pltpu.async_copy(
            table_hbm.at[idx_v.at[pl.ds(c * _C, _C)]], rows_v.at[b], gsem.at[b]
        )

    def store_handle(c):
        b = c % _NBUF
        return pltpu.make_async_copy(
            rows_v.at[b], out_hbm.at[pl.ds(base + c * _C, _C)], ssem.at[b]
        )

    for c in range(_NBUF - 1):
        gather(c)
    for c in range(_NCHUNK):
        b = c % _NBUF
        pltpu.make_async_copy(
            table_hbm.at[idx_v.at[pl.ds(c * _C, _C)]], rows_v.at[b], gsem.at[b]
        ).wait()
        store_handle(c).start()
        g = c + _NBUF - 1
        if g < _NCHUNK:
            if g - _NBUF >= 0:
                store_handle(g - _NBUF).wait()
            gather(g)
    for c in range(_NCHUNK - _NBUF, _NCHUNK):
        store_handle(c).wait()


def kernel(pos_seq, pos_emb):
    idx = pos_seq.reshape(-1).astype(jnp.int32)
    out = _emb_gather(idx, pos_emb)
    return out.reshape(pos_seq.shape + (pos_emb.shape[-1],))


# speedup vs baseline: 2.4770x; 1.0003x over previous
import functools

import jax
import jax.numpy as jnp
from jax import lax
from jax.experimental import pallas as pl
from jax.experimental.pallas import tpu as pltpu
from jax.experimental.pallas import tpu_sc as plsc

_DEMB = 768
_NC = 2
_NS = 16
_NW = _NC * _NS
_B = 32768
_BPW = _B // _NW
_C = 16
_NBUF = 8
_NCHUNK = _BPW // _C

_mesh = plsc.VectorSubcoreMesh(core_axis_name="c", subcore_axis_name="s")


@functools.partial(
    pl.kernel,
    out_type=jax.ShapeDtypeStruct((_B, _DEMB), jnp.float32),
    mesh=_mesh,
    scratch_types=[
        pltpu.VMEM((_BPW,), jnp.int32),
        pltpu.VMEM((_NBUF, _C, _DEMB), jnp.float32),
        pltpu.SemaphoreType.DMA((_NBUF,)),
        pltpu.SemaphoreType.DMA((_NBUF,)),
    ],
)
def _emb_gather(idx_hbm, table_hbm, out_hbm, idx_v, rows_v, gsem, ssem):
    wid = lax.axis_index("s") * _NC + lax.axis_index("c")
    base = wid * _BPW
    pltpu.sync_copy(idx_hbm.at[pl.ds(base, _BPW)], idx_v)

    def gather(c):
        b = c % _NBUF
        pltpu.async_copy(
            table_hbm.at[idx_v.at[pl.ds(c * _C, _C)]], rows_v.at[b], gsem.at[b]
        )

    def store_handle(c):
        b = c % _NBUF
        return pltpu.make_async_copy(
            rows_v.at[b], out_hbm.at[pl.ds(base + c * _C, _C)], ssem.at[b]
        )

    for c in range(_NBUF - 1):
        gather(c)
    for c in range(_NCHUNK):
        b = c % _NBUF
        pltpu.make_async_copy(
            table_hbm.at[idx_v.at[pl.ds(c * _C, _C)]], rows_v.at[b], gsem.at[b]
        ).wait()
        store_handle(c).start()
        g = c + _NBUF - 1
        if g < _NCHUNK:
            if g - _NBUF >= 0:
                store_handle(g - _NBUF).wait()
            gather(g)
    for c in range(_NCHUNK - _NBUF, _NCHUNK):
        store_handle(c).wait()


def kernel(pos_seq, pos_emb):
    idx = pos_seq.reshape(-1).astype(jnp.int32)
    out = _emb_gather(idx, pos_emb)
    return out.reshape(pos_seq.shape + (pos_emb.shape[-1],))
